# baseline (speedup 1.0000x reference)
import functools

import jax
import jax.numpy as jnp
from jax.experimental import pallas as pl
from jax.experimental.pallas import tpu as pltpu

_ROWS = 16384
_D_IN = 512
_D_HID = 128
_BLOCK_R = 2048


def _fused_ae_kernel(x_ref, we_ref, be_ref, wd_ref, bd_ref, out_ref,
                     w_ref, c_ref):
    @pl.when(pl.program_id(0) == 0)
    def _init():
        w_ref[...] = jnp.dot(we_ref[...].astype(jnp.bfloat16),
                             wd_ref[...].astype(jnp.bfloat16),
                             preferred_element_type=jnp.float32
                             ).astype(jnp.bfloat16)
        c_ref[...] = jnp.dot(be_ref[...], wd_ref[...],
                             preferred_element_type=jnp.float32) + bd_ref[...]

    r = jnp.dot(x_ref[...].astype(jnp.bfloat16), w_ref[...],
                preferred_element_type=jnp.float32)
    out_ref[...] = r + c_ref[...]


@functools.partial(jax.jit, static_argnames=())
def kernel(x, We, be, Wd, bd):
    be2 = be.reshape(1, _D_HID)
    bd2 = bd.reshape(1, _D_IN)
    grid = (_ROWS // _BLOCK_R,)
    return pl.pallas_call(
        _fused_ae_kernel,
        grid=grid,
        in_specs=[
            pl.BlockSpec((_BLOCK_R, _D_IN), lambda i: (i, 0)),
            pl.BlockSpec((_D_IN, _D_HID), lambda i: (0, 0)),
            pl.BlockSpec((1, _D_HID), lambda i: (0, 0)),
            pl.BlockSpec((_D_HID, _D_IN), lambda i: (0, 0)),
            pl.BlockSpec((1, _D_IN), lambda i: (0, 0)),
        ],
        out_specs=pl.BlockSpec((_BLOCK_R, _D_IN), lambda i: (i, 0)),
        out_shape=jax.ShapeDtypeStruct((_ROWS, _D_IN), jnp.float32),
        scratch_shapes=[
            pltpu.VMEM((_D_IN, _D_IN), jnp.bfloat16),
            pltpu.VMEM((1, _D_IN), jnp.float32),
        ],
    )(x, We, be2, Wd, bd2)


# speedup vs baseline: 1.1653x; 1.1653x over previous
import functools

import jax
import jax.numpy as jnp
from jax.experimental import pallas as pl
from jax.experimental.pallas import tpu as pltpu

_ROWS = 16384
_D_IN = 512
_D_HID = 128
_CHUNK = 2048
_NCHUNK = _ROWS // _CHUNK
_NBUF = 4


def _manual_kernel(x_hbm, we_ref, be_ref, wd_ref, bd_ref, out_hbm,
                   xbuf, obuf, w_ref, c_ref, in_sems, out_sems):
    w_ref[...] = jnp.dot(we_ref[...].astype(jnp.bfloat16),
                         wd_ref[...].astype(jnp.bfloat16),
                         preferred_element_type=jnp.float32
                         ).astype(jnp.bfloat16)
    c_ref[...] = jnp.dot(be_ref[...], wd_ref[...],
                         preferred_element_type=jnp.float32) + bd_ref[...]

    def in_copy(chunk, slot):
        return pltpu.make_async_copy(
            x_hbm.at[pl.ds(chunk * _CHUNK, _CHUNK), :],
            xbuf.at[slot], in_sems.at[slot])

    def out_copy(chunk, slot):
        return pltpu.make_async_copy(
            obuf.at[slot],
            out_hbm.at[pl.ds(chunk * _CHUNK, _CHUNK), :], out_sems.at[slot])

    for i in range(_NBUF):
        in_copy(i, i).start()

    for i in range(_NCHUNK):
        slot = i % _NBUF
        in_copy(i, slot).wait()
        if i >= _NBUF:
            out_copy(i - _NBUF, slot).wait()
        obuf[slot] = jnp.dot(xbuf[slot].astype(jnp.bfloat16), w_ref[...],
                             preferred_element_type=jnp.float32) + c_ref[...]
        out_copy(i, slot).start()
        if i + _NBUF < _NCHUNK:
            in_copy(i + _NBUF, slot).start()

    for i in range(max(0, _NCHUNK - _NBUF), _NCHUNK):
        out_copy(i, i % _NBUF).wait()


@functools.partial(jax.jit, static_argnames=())
def kernel(x, We, be, Wd, bd):
    be2 = be.reshape(1, _D_HID)
    bd2 = bd.reshape(1, _D_IN)
    return pl.pallas_call(
        _manual_kernel,
        in_specs=[
            pl.BlockSpec(memory_space=pl.MemorySpace.ANY),
            pl.BlockSpec(memory_space=pltpu.MemorySpace.VMEM),
            pl.BlockSpec(memory_space=pltpu.MemorySpace.VMEM),
            pl.BlockSpec(memory_space=pltpu.MemorySpace.VMEM),
            pl.BlockSpec(memory_space=pltpu.MemorySpace.VMEM),
        ],
        out_specs=pl.BlockSpec(memory_space=pl.MemorySpace.ANY),
        out_shape=jax.ShapeDtypeStruct((_ROWS, _D_IN), jnp.float32),
        scratch_shapes=[
            pltpu.VMEM((_NBUF, _CHUNK, _D_IN), jnp.float32),
            pltpu.VMEM((_NBUF, _CHUNK, _D_IN), jnp.float32),
            pltpu.VMEM((_D_IN, _D_IN), jnp.bfloat16),
            pltpu.VMEM((1, _D_IN), jnp.float32),
            pltpu.SemaphoreType.DMA((_NBUF,)),
            pltpu.SemaphoreType.DMA((_NBUF,)),
        ],
    )(x, We, be2, Wd, bd2)
